# baseline (speedup 1.0000x reference)
import functools

import jax
import jax.numpy as jnp
from jax import lax
from jax.experimental import pallas as pl
from jax.experimental.pallas import tpu as pltpu
from jax.experimental.pallas import tpu_sc as plsc

N_NODES = 10000
NPAD = 10240
E = 320000
NCH = 160
EPT = NCH * 128
EPAD = EPT * 16
SLC = 640
SRC_SH = 14
M14 = (1 << 14) - 1
FBIT = 1 << 28
CCAP = EPAD + 2048
CZ = CCAP // 16 // 8
WV = 16
WV7 = 8


def _sc_mesh():
    return plsc.VectorSubcoreMesh(core_axis_name="c", subcore_axis_name="s",
                                  num_cores=1)


def _prep(pk3, ewp3):
    f32 = jnp.float32
    i32 = jnp.int32
    out_type = [
        jax.ShapeDtypeStruct((NPAD,), f32),
        jax.ShapeDtypeStruct((NPAD,), f32),
        jax.ShapeDtypeStruct((CCAP,), i32),
        jax.ShapeDtypeStruct((CCAP,), f32),
        jax.ShapeDtypeStruct((16, 16), i32),
        jax.ShapeDtypeStruct((NPAD,), f32),
        jax.ShapeDtypeStruct((NPAD,), f32),
    ]
    scratch = [
        pltpu.VMEM((NCH, 128), i32),
        pltpu.VMEM((NCH, 128), i32),
        pltpu.VMEM((NCH, 128), f32),
        pltpu.VMEM((NCH, 128), f32),
        pltpu.VMEM((NCH, 128), f32),
        pltpu.VMEM((SLC,), f32),
        pltpu.VMEM((SLC,), f32),
        pltpu.VMEM((16,), i32),
        pltpu.VMEM((16,), f32),
        pltpu.VMEM((128,), f32),
        pltpu.VMEM_SHARED((NPAD,), f32),
        pltpu.VMEM_SHARED((NPAD,), f32),
        pltpu.VMEM_SHARED((NPAD,), f32),
        pltpu.SemaphoreType.DMA,
    ]

    @functools.partial(pl.kernel, out_type=out_type, mesh=_sc_mesh(),
                       scratch_types=scratch)
    def k(pk_h, ewp_h, dinv_h, a2_h, e1p_h, e1n_h, cnt_h, m1_h, m2_h,
          srcV, dstV, nrmV, dsV, ddV,
          sliceA, sliceB, cntT, d0T, g7B,
          degS, a2S, m1S, sem):
        tid = lax.axis_index("s")
        nsl = pl.ds(tid * SLC, SLC)
        ones16 = jnp.ones((16,), f32)
        zf16 = jnp.zeros((16,), f32)

        pltpu.sync_copy(pk_h.at[tid], srcV)
        pltpu.sync_copy(ewp_h.at[tid], nrmV)

        def _fill(i, _):
            sliceA[pl.ds(i * 16, 16)] = ones16
            sliceB[pl.ds(i * 16, 16)] = zf16
            return 0
        lax.fori_loop(0, SLC // 16, _fill, 0)
        pltpu.sync_copy(sliceA, degS.at[nsl])
        pltpu.sync_copy(sliceB, a2S.at[nsl])
        pltpu.sync_copy(sliceB, m1S.at[nsl])

        plsc.subcore_barrier()

        def _p1(j, _):
            for kk in range(8):
                d = pl.ds(kk * 16, 16)
                v = srcV[j, d]
                dstV[j, d] = v & M14
                srcV[j, d] = lax.shift_right_logical(v, SRC_SH)
                p = nrmV[j, d]
                w = 1.0 / (1.0 + jnp.exp(-p))
                nrmV[j, d] = w
                dsV[j, d] = jnp.where(dstV[j, d] == 0, w, 0.0)
            pltpu.sync_copy(nrmV.at[j], degS.at[dstV.at[j]], add=True)
            pltpu.sync_copy(dsV.at[j], a2S.at[srcV.at[j]], add=True)
            return 0
        lax.fori_loop(0, NCH, _p1, 0)
        plsc.subcore_barrier()

        pltpu.sync_copy(degS.at[nsl], sliceA)

        def _p2(i, _):
            d = pl.ds(i * 16, 16)
            x = sliceA[d]
            iv = lax.bitcast_convert_type(x, i32)
            iv = 0x5F3759DF - lax.shift_right_arithmetic(iv, 1)
            y = lax.bitcast_convert_type(iv, f32)
            y = y * (1.5 - 0.5 * x * y * y)
            y = y * (1.5 - 0.5 * x * y * y)
            y = y * (1.5 - 0.5 * x * y * y)
            sliceA[d] = y
            return 0
        lax.fori_loop(0, SLC // 16, _p2, 0)
        pltpu.sync_copy(sliceA, dinv_h.at[nsl])
        pltpu.sync_copy(sliceA, degS.at[nsl])
        plsc.subcore_barrier()

        pltpu.sync_copy(degS.at[pl.ds(0, 16)], d0T)
        d0 = d0T[pl.ds(0, 16)][0]
        pltpu.sync_copy(a2S.at[nsl], sliceA)
        pltpu.sync_copy(degS.at[nsl], sliceB)

        def _p4(i, _):
            d = pl.ds(i * 16, 16)
            vid = lax.iota(i32, 16) + (tid * SLC + i * 16)
            a = sliceA[d] * sliceB[d] * d0
            sliceA[d] = a
            sliceB[d] = jnp.where((a > 0.0) | (vid == 0), 1.0, 0.0)
            return 0
        lax.fori_loop(0, SLC // 16, _p4, 0)
        pltpu.sync_copy(sliceA, a2_h.at[nsl])
        pltpu.sync_copy(sliceB, m2_h.at[nsl])
        plsc.subcore_barrier()

        def _p5(j, _):
            pltpu.sync_copy(m2_h.at[dstV.at[j]], ddV.at[j])
            pltpu.sync_copy(ddV.at[j], m1S.at[srcV.at[j]], add=True)
            return 0
        lax.fori_loop(0, NCH, _p5, 0)
        plsc.subcore_barrier()

        pltpu.sync_copy(m1S.at[nsl], sliceA)
        pltpu.sync_copy(m2_h.at[nsl], sliceB)

        def _p6(i, _):
            d = pl.ds(i * 16, 16)
            sliceA[d] = sliceA[d] + sliceB[d]
            return 0
        lax.fori_loop(0, SLC // 16, _p6, 0)
        pltpu.sync_copy(sliceA, m1_h.at[nsl])
        plsc.subcore_barrier()

        lane = lax.iota(i32, 16)
        gdn = lax.GatherDimensionNumbers(offset_dims=(),
                                         collapsed_slice_dims=(0,),
                                         start_index_map=(0,))

        def _p7(j, cnt):
            pltpu.sync_copy(m1_h.at[dstV.at[j]], g7B)
            for kk in range(8):
                d = pl.ds(kk * 16, 16)
                act = g7B[d] > 0.0
                cs = jnp.where(act, 1, 0)
                for sh in (1, 2, 4, 8):
                    g = lax.gather(cs,
                                   jnp.maximum(lane - sh, 0).reshape(16, 1),
                                   gdn, (1,),
                                   mode=lax.GatherScatterMode.PROMISE_IN_BOUNDS)
                    cs = cs + jnp.where(lane >= sh, g, 0)
                dump = EPAD + (j % 16) * 128 + kk * 16 + lane
                pk = srcV[j, d] * (M14 + 1) + dstV[j, d]
                srcV[j, d] = pk + jnp.where(ddV[j, d] > 0.0, FBIT, 0)
                dstV[j, d] = jnp.where(act, tid * EPT + cnt + cs - 1, dump)
                cnt = cnt + cs[15]
            pltpu.sync_copy(srcV.at[j], e1p_h.at[dstV.at[j]])
            pltpu.sync_copy(nrmV.at[j], e1n_h.at[dstV.at[j]])
            return cnt
        cnt = lax.fori_loop(0, NCH, _p7, jnp.int32(0))


        cntT[pl.ds(0, 16)] = jnp.full((16,), cnt, i32)
        pltpu.sync_copy(cntT, cnt_h.at[tid])

    return k(pk3, ewp3)


def _edge_layer(h, e1p3, e1n3, cnt, dinv, layer2):
    f32 = jnp.float32
    i32 = jnp.int32
    scratch = [
        pltpu.VMEM((2, 128), i32),
        pltpu.VMEM((2, 128), i32),
        pltpu.VMEM((2, 128), i32),
        pltpu.VMEM((144,), f32),
        pltpu.VMEM((144,), f32),
        pltpu.VMEM((128,), f32),
        pltpu.VMEM((128,), f32),
        pltpu.VMEM((1, 128, 128), f32),
        pltpu.VMEM((16,), i32),
        pltpu.VMEM_SHARED((NPAD, 128), f32),
        pltpu.SemaphoreType.DMA,
    ]

    @functools.partial(pl.kernel,
                       out_type=[jax.ShapeDtypeStruct((NPAD, 128), f32)],
                       mesh=_sc_mesh(), scratch_types=scratch)
    def k(h_h, e1p_h, e1n_h, cnt_h, dinv_h, acc_h,
          pkC, srcC, idxW, nwA, nwB, dsg, ddg, rowsB, cntT, accS, sem):
        nw = [nwA, nwB]
        tid = lax.axis_index("s")
        zf16 = jnp.zeros((16,), f32)

        def _zr(r, _):
            for kk in range(8):
                rowsB[0, r, pl.ds(kk * 16, 16)] = zf16
            return 0
        lax.fori_loop(0, 128, _zr, 0)
        for m in range(SLC // 128):
            pltpu.sync_copy(rowsB.at[0],
                            accS.at[pl.ds(tid * SLC + m * 128, 128)])
        pltpu.sync_copy(cnt_h.at[tid], cntT)
        plsc.subcore_barrier()

        cnt = cntT[pl.ds(0, 16)][0]

        lane = lax.iota(i32, 16)

        def _chunk(j, _):
            base = tid * EPT + j * 128
            pltpu.sync_copy(e1p_h.at[pl.ds(base, 128)], pkC.at[0])
            pltpu.sync_copy(e1n_h.at[pl.ds(base, 128)],
                            nw[0].at[pl.ds(0, 128)])
            for kk in range(8):
                d = pl.ds(kk * 16, 16)
                valid = (j * 128 + kk * 16 + lane) < cnt
                v = jnp.where(valid, pkC[0, d], 0)
                srcC[0, d] = lax.shift_right_logical(v, SRC_SH) & M14
                idxW[0, d] = v & M14
                nv = jnp.where(valid, nw[0][d], 0.0)
                if layer2:
                    nv = jnp.where(v >= FBIT, nv, 0.0)
                nw[0][d] = nv
            pltpu.sync_copy(dinv_h.at[srcC.at[0]], dsg)
            pltpu.sync_copy(dinv_h.at[idxW.at[0]], ddg)
            for kk in range(8):
                d = pl.ds(kk * 16, 16)
                nw[0][d] = nw[0][d] * dsg[d] * ddg[d]
            pltpu.sync_copy(h_h.at[srcC.at[0]], rowsB.at[0])

            def _scale(r, _):
                s = nw[0][pl.ds(r, 16)][0]
                for kk in range(8):
                    d = pl.ds(kk * 16, 16)
                    rowsB[0, r, d] = rowsB[0, r, d] * s
                return 0
            lax.fori_loop(0, 128, _scale, 0)
            pltpu.sync_copy(rowsB.at[0], accS.at[idxW.at[0]], add=True)
            return 0
        nch = (cnt + 127) // 128
        lax.fori_loop(0, nch, _chunk, 0)
        plsc.subcore_barrier()

        for m in range(SLC // 128):
            rs = pl.ds(tid * SLC + m * 128, 128)
            pltpu.sync_copy(accS.at[rs], acc_h.at[rs])

    return k(h, e1p3, e1n3, cnt, dinv)[0]


def _mm_kernel(x_ref, w_ref, o_ref):
    o_ref[...] = jnp.dot(x_ref[...], w_ref[...],
                         preferred_element_type=jnp.float32)


def _tc_matmul(x, W):
    n, d = x.shape
    dout = W.shape[1]
    blk = 1280
    return pl.pallas_call(
        _mm_kernel,
        grid=(n // blk,),
        in_specs=[pl.BlockSpec((blk, d), lambda i: (i, 0)),
                  pl.BlockSpec((d, dout), lambda i: (0, 0))],
        out_specs=pl.BlockSpec((blk, dout), lambda i: (i, 0)),
        out_shape=jax.ShapeDtypeStruct((n, dout), jnp.float32),
    )(x, W)


def _mid_kernel(acc_ref, hp_ref, di_ref, b_ref, w_ref, o_ref):
    d2 = di_ref[...] * di_ref[...]
    z = jnp.maximum(acc_ref[...] + d2 * hp_ref[...] + b_ref[...], 0.0)
    o_ref[...] = jnp.dot(z, w_ref[...], preferred_element_type=jnp.float32)


def _tc_mid(acc, hp, dinv, b, W):
    n, d = acc.shape
    blk = 1280
    return pl.pallas_call(
        _mid_kernel,
        grid=(n // blk,),
        in_specs=[pl.BlockSpec((blk, d), lambda i: (i, 0)),
                  pl.BlockSpec((blk, d), lambda i: (i, 0)),
                  pl.BlockSpec((blk, 1), lambda i: (i, 0)),
                  pl.BlockSpec((1, d), lambda i: (0, 0)),
                  pl.BlockSpec((d, d), lambda i: (0, 0))],
        out_specs=pl.BlockSpec((blk, d), lambda i: (i, 0)),
        out_shape=jax.ShapeDtypeStruct((n, d), jnp.float32),
    )(acc, hp, dinv, b, W)


def _fin_kernel(acc_ref, hp_ref, di_ref, b_ref, a2_ref, w3_ref, b3_ref,
                o_ref, scr_ref):
    i = pl.program_id(0)
    d2 = di_ref[...] * di_ref[...]
    z = jnp.maximum(acc_ref[...] + d2 * hp_ref[...] + b_ref[...], 0.0)
    pv = jnp.sum(a2_ref[...] * z, axis=0, keepdims=True)

    @pl.when(i == 0)
    def _():
        scr_ref[...] = pv

    @pl.when(i > 0)
    def _():
        scr_ref[...] = scr_ref[...] + pv

    @pl.when(i == pl.num_programs(0) - 1)
    def _():
        logits = jnp.dot(scr_ref[...], w3_ref[...],
                         preferred_element_type=jnp.float32) + b3_ref[...]
        mx = jnp.max(logits)
        ls = logits - mx
        o_ref[...] = ls - jnp.log(jnp.sum(jnp.exp(ls)))


def _tc_fin(acc, hp, dinv, b, a2c, W3, b3):
    n, d = acc.shape
    dout = W3.shape[1]
    blk = 1280
    return pl.pallas_call(
        _fin_kernel,
        grid=(n // blk,),
        in_specs=[pl.BlockSpec((blk, d), lambda i: (i, 0)),
                  pl.BlockSpec((blk, d), lambda i: (i, 0)),
                  pl.BlockSpec((blk, 1), lambda i: (i, 0)),
                  pl.BlockSpec((1, d), lambda i: (0, 0)),
                  pl.BlockSpec((blk, 1), lambda i: (i, 0)),
                  pl.BlockSpec((d, dout), lambda i: (0, 0)),
                  pl.BlockSpec((1, dout), lambda i: (0, 0))],
        out_specs=pl.BlockSpec((1, dout), lambda i: (0, 0)),
        out_shape=jax.ShapeDtypeStruct((1, dout), jnp.float32),
        scratch_shapes=[pltpu.VMEM((1, d), jnp.float32)],
    )(acc, hp, dinv, b, a2c, W3, b3)


def kernel(x, edge_index, edge_weight_params, W1, b1, W2, b2, W3, b3):
    i32 = jnp.int32
    f32 = jnp.float32
    npe = EPAD - E
    pk = edge_index[0].astype(i32) * (M14 + 1) + edge_index[1].astype(i32)
    pk = jnp.concatenate([pk, jnp.zeros((npe,), i32)])
    ewp = jnp.concatenate([edge_weight_params.astype(f32),
                           jnp.full((npe,), -1e9, f32)])
    pk3 = pk.reshape(16, NCH, 128)
    ewp3 = ewp.reshape(16, NCH, 128)

    dinv, a2, e1p, e1n, cnt, _m1, _m2 = _prep(pk3, ewp3)
    e1p3 = e1p[:EPAD]
    e1n3 = e1n[:EPAD]

    xpad = jnp.pad(x, ((0, NPAD - N_NODES), (0, 0)))
    h1p = _tc_matmul(xpad, W1)

    acc1 = _edge_layer(h1p, e1p3, e1n3, cnt, dinv, layer2=False)
    dcol = dinv.reshape(NPAD, 1)
    g2 = _tc_mid(acc1, h1p, dcol, b1.reshape(1, -1), W2)

    acc2 = _edge_layer(g2, e1p3, e1n3, cnt, dinv, layer2=True)
    a2p = a2.at[0].add(dinv[0] * dinv[0])
    logp = _tc_fin(acc2, g2, dcol, b2.reshape(1, -1), a2p.reshape(NPAD, 1),
                   W3, b3.reshape(1, -1))
    return logp[0]


# speedup vs baseline: 3.9964x; 3.9964x over previous
import functools

import jax
import jax.numpy as jnp
from jax import lax
from jax.experimental import pallas as pl
from jax.experimental.pallas import tpu as pltpu
from jax.experimental.pallas import tpu_sc as plsc

N_NODES = 10000
NPAD = 10240
E = 320000
NCH = 160
EPT = NCH * 128
EPAD = EPT * 16
SLC = 640
SRC_SH = 14
M14 = (1 << 14) - 1
FBIT = 1 << 28
CCAP = EPAD + 2048
CZ = CCAP // 16 // 8
WV = 16
WV7 = 8


def _sc_mesh():
    return plsc.VectorSubcoreMesh(core_axis_name="c", subcore_axis_name="s",
                                  num_cores=1)


def _prep(pk3, ewp3):
    f32 = jnp.float32
    i32 = jnp.int32
    out_type = [
        jax.ShapeDtypeStruct((NPAD,), f32),
        jax.ShapeDtypeStruct((NPAD,), f32),
        jax.ShapeDtypeStruct((CCAP,), i32),
        jax.ShapeDtypeStruct((CCAP,), i32),
        jax.ShapeDtypeStruct((16, 16), i32),
        jax.ShapeDtypeStruct((NPAD,), f32),
        jax.ShapeDtypeStruct((NPAD,), f32),
    ]
    scratch = [
        pltpu.VMEM((NCH, 128), i32),
        pltpu.VMEM((NCH, 128), i32),
        pltpu.VMEM((NCH, 128), f32),
        pltpu.VMEM((NCH, 128), f32),
        pltpu.VMEM((NCH, 128), f32),
        pltpu.VMEM((SLC,), f32),
        pltpu.VMEM((SLC,), f32),
        pltpu.VMEM((16,), i32),
        pltpu.VMEM((16,), f32),
        pltpu.VMEM((128,), f32),
        pltpu.VMEM_SHARED((NPAD,), f32),
        pltpu.VMEM_SHARED((NPAD,), f32),
        pltpu.VMEM_SHARED((NPAD,), f32),
        pltpu.VMEM_SHARED((CCAP,), i32),
        pltpu.SemaphoreType.DMA,
    ]

    @functools.partial(pl.kernel, out_type=out_type, mesh=_sc_mesh(),
                       scratch_types=scratch)
    def k(pk_h, ewp_h, dinv_h, a2_h, e1p_h, e1n_h, cnt_h, m1_h, m2_h,
          srcV, dstV, nrmV, dsV, ddV,
          sliceA, sliceB, cntT, d0T, g7B,
          degS, a2S, m1S, cS, sem):
        tid = lax.axis_index("s")
        nsl = pl.ds(tid * SLC, SLC)
        ones16 = jnp.ones((16,), f32)
        zf16 = jnp.zeros((16,), f32)

        pltpu.sync_copy(pk_h.at[pl.ds(tid * NCH, NCH)], srcV)
        pltpu.sync_copy(ewp_h.at[pl.ds(tid * NCH, NCH)], nrmV)

        def _fill(i, _):
            sliceA[pl.ds(i * 16, 16)] = ones16
            sliceB[pl.ds(i * 16, 16)] = zf16
            return 0
        lax.fori_loop(0, SLC // 16, _fill, 0)
        pltpu.sync_copy(sliceA, degS.at[nsl])
        pltpu.sync_copy(sliceB, a2S.at[nsl])
        pltpu.sync_copy(sliceB, m1S.at[nsl])

        plsc.subcore_barrier()

        def _p1(j, _):
            for kk in range(8):
                d = pl.ds(kk * 16, 16)
                v = srcV[j, d]
                dstV[j, d] = v & M14
                srcV[j, d] = lax.shift_right_logical(v, SRC_SH)
                p = nrmV[j, d]
                w = 1.0 / (1.0 + jnp.exp(-p))
                nrmV[j, d] = w
                dsV[j, d] = jnp.where(dstV[j, d] == 0, w, 0.0)
            pltpu.sync_copy(nrmV.at[j], degS.at[dstV.at[j]], add=True)
            pltpu.sync_copy(dsV.at[j], a2S.at[srcV.at[j]], add=True)
            return 0
        lax.fori_loop(0, NCH, _p1, 0)
        plsc.subcore_barrier()

        pltpu.sync_copy(degS.at[nsl], sliceA)

        def _p2(i, _):
            d = pl.ds(i * 16, 16)
            x = sliceA[d]
            iv = lax.bitcast_convert_type(x, i32)
            iv = 0x5F3759DF - lax.shift_right_arithmetic(iv, 1)
            y = lax.bitcast_convert_type(iv, f32)
            y = y * (1.5 - 0.5 * x * y * y)
            y = y * (1.5 - 0.5 * x * y * y)
            y = y * (1.5 - 0.5 * x * y * y)
            sliceA[d] = y
            return 0
        lax.fori_loop(0, SLC // 16, _p2, 0)
        pltpu.sync_copy(sliceA, dinv_h.at[nsl])
        pltpu.sync_copy(sliceA, degS.at[nsl])
        plsc.subcore_barrier()

        pltpu.sync_copy(degS.at[pl.ds(0, 16)], d0T)
        d0 = d0T[pl.ds(0, 16)][0]
        pltpu.sync_copy(a2S.at[nsl], sliceA)
        pltpu.sync_copy(degS.at[nsl], sliceB)

        def _p4(i, _):
            d = pl.ds(i * 16, 16)
            vid = lax.iota(i32, 16) + (tid * SLC + i * 16)
            a = sliceA[d] * sliceB[d] * d0
            sliceA[d] = a
            sliceB[d] = jnp.where((a > 0.0) | (vid == 0), 1.0, 0.0)
            return 0
        lax.fori_loop(0, SLC // 16, _p4, 0)
        pltpu.sync_copy(sliceA, a2_h.at[nsl])
        pltpu.sync_copy(sliceB, m2_h.at[nsl])
        plsc.subcore_barrier()

        def _p5(j, _):
            pltpu.sync_copy(m2_h.at[dstV.at[j]], ddV.at[j])
            pltpu.sync_copy(ddV.at[j], m1S.at[srcV.at[j]], add=True)
            return 0
        lax.fori_loop(0, NCH, _p5, 0)
        plsc.subcore_barrier()

        pltpu.sync_copy(m1S.at[nsl], sliceA)
        pltpu.sync_copy(m2_h.at[nsl], sliceB)

        def _p6(i, _):
            d = pl.ds(i * 16, 16)
            sliceA[d] = sliceA[d] + sliceB[d]
            return 0
        lax.fori_loop(0, SLC // 16, _p6, 0)
        pltpu.sync_copy(sliceA, m1_h.at[nsl])
        plsc.subcore_barrier()

        lane = lax.iota(i32, 16)
        gdn = lax.GatherDimensionNumbers(offset_dims=(),
                                         collapsed_slice_dims=(0,),
                                         start_index_map=(0,))

        def _p7(j, cnt):
            pltpu.sync_copy(m1_h.at[dstV.at[j]], g7B)
            for kk in range(8):
                d = pl.ds(kk * 16, 16)
                act = g7B[d] > 0.0
                cs = jnp.where(act, 1, 0)
                for sh in (1, 2, 4, 8):
                    g = lax.gather(cs,
                                   jnp.maximum(lane - sh, 0).reshape(16, 1),
                                   gdn, (1,),
                                   mode=lax.GatherScatterMode.PROMISE_IN_BOUNDS)
                    cs = cs + jnp.where(lane >= sh, g, 0)
                dump = EPAD + (j % 16) * 128 + kk * 16 + lane
                pk = srcV[j, d] * (M14 + 1) + dstV[j, d]
                srcV[j, d] = pk + jnp.where(ddV[j, d] > 0.0, FBIT, 0)
                dstV[j, d] = jnp.where(act, tid * EPT + cnt + cs - 1, dump)
                cnt = cnt + cs[15]
            pltpu.sync_copy(srcV.at[j], cS.at[dstV.at[j]])
            return cnt
        cnt = lax.fori_loop(0, NCH, _p7, jnp.int32(0))

        esl = pl.ds(tid * EPT, EPT)
        pltpu.sync_copy(cS.at[esl], e1p_h.at[esl])

        def _p8(j, _):
            for kk in range(8):
                d = pl.ds(kk * 16, 16)
                srcV[j, d] = lax.bitcast_convert_type(nrmV[j, d], i32)
            pltpu.sync_copy(srcV.at[j], cS.at[dstV.at[j]])
            return 0
        lax.fori_loop(0, NCH, _p8, 0)
        pltpu.sync_copy(cS.at[esl], e1n_h.at[esl])


        cntT[pl.ds(0, 16)] = jnp.full((16,), cnt, i32)
        pltpu.sync_copy(cntT, cnt_h.at[tid])

    return k(pk3, ewp3)


def _edge_layer(h, e1p3, e1n3, cnt, dinv, layer2):
    f32 = jnp.float32
    i32 = jnp.int32
    scratch = [
        pltpu.VMEM((2, 128), i32),
        pltpu.VMEM((2, 128), i32),
        pltpu.VMEM((2, 128), i32),
        pltpu.VMEM((144,), f32),
        pltpu.VMEM((144,), f32),
        pltpu.VMEM((128,), f32),
        pltpu.VMEM((128,), f32),
        pltpu.VMEM((1, 128, 128), f32),
        pltpu.VMEM((16,), i32),
        pltpu.VMEM_SHARED((NPAD, 128), f32),
        pltpu.SemaphoreType.DMA,
    ]

    @functools.partial(pl.kernel,
                       out_type=[jax.ShapeDtypeStruct((NPAD, 128), f32)],
                       mesh=_sc_mesh(), scratch_types=scratch)
    def k(h_h, e1p_h, e1n_h, cnt_h, dinv_h, acc_h,
          pkC, srcC, idxW, nwA, nwB, dsg, ddg, rowsB, cntT, accS, sem):
        nw = [nwA, nwB]
        tid = lax.axis_index("s")
        zf16 = jnp.zeros((16,), f32)

        def _zr(r, _):
            for kk in range(8):
                rowsB[0, r, pl.ds(kk * 16, 16)] = zf16
            return 0
        lax.fori_loop(0, 128, _zr, 0)
        for m in range(SLC // 128):
            pltpu.sync_copy(rowsB.at[0],
                            accS.at[pl.ds(tid * SLC + m * 128, 128)])
        pltpu.sync_copy(cnt_h.at[tid], cntT)
        plsc.subcore_barrier()

        cnt = cntT[pl.ds(0, 16)][0]

        lane = lax.iota(i32, 16)

        def _chunk(j, _):
            base = tid * EPT + j * 128
            pltpu.sync_copy(e1p_h.at[pl.ds(base, 128)], pkC.at[0])
            pltpu.sync_copy(e1n_h.at[pl.ds(base, 128)],
                            nw[0].at[pl.ds(0, 128)])
            for kk in range(8):
                d = pl.ds(kk * 16, 16)
                valid = (j * 128 + kk * 16 + lane) < cnt
                v = jnp.where(valid, pkC[0, d], 0)
                srcC[0, d] = lax.shift_right_logical(v, SRC_SH) & M14
                idxW[0, d] = v & M14
                nv = jnp.where(valid, nw[0][d], 0.0)
                if layer2:
                    nv = jnp.where(v >= FBIT, nv, 0.0)
                nw[0][d] = nv
            pltpu.sync_copy(dinv_h.at[srcC.at[0]], dsg)
            pltpu.sync_copy(dinv_h.at[idxW.at[0]], ddg)
            for kk in range(8):
                d = pl.ds(kk * 16, 16)
                nw[0][d] = nw[0][d] * dsg[d] * ddg[d]
            pltpu.sync_copy(h_h.at[srcC.at[0]], rowsB.at[0])

            def _scale(r, _):
                s = nw[0][pl.ds(r, 16)][0]
                for kk in range(8):
                    d = pl.ds(kk * 16, 16)
                    rowsB[0, r, d] = rowsB[0, r, d] * s
                return 0
            lax.fori_loop(0, 128, _scale, 0)
            pltpu.sync_copy(rowsB.at[0], accS.at[idxW.at[0]], add=True)
            return 0
        nch = (cnt + 127) // 128
        lax.fori_loop(0, nch, _chunk, 0)
        plsc.subcore_barrier()

        for m in range(SLC // 128):
            rs = pl.ds(tid * SLC + m * 128, 128)
            pltpu.sync_copy(accS.at[rs], acc_h.at[rs])

    return k(h, e1p3, e1n3, cnt, dinv)[0]


def _mm_kernel(x_ref, w_ref, o_ref):
    o_ref[...] = jnp.dot(x_ref[...], w_ref[...],
                         preferred_element_type=jnp.float32)


def _tc_matmul(x, W):
    n, d = x.shape
    dout = W.shape[1]
    blk = 1280
    return pl.pallas_call(
        _mm_kernel,
        grid=(n // blk,),
        in_specs=[pl.BlockSpec((blk, d), lambda i: (i, 0)),
                  pl.BlockSpec((d, dout), lambda i: (0, 0))],
        out_specs=pl.BlockSpec((blk, dout), lambda i: (i, 0)),
        out_shape=jax.ShapeDtypeStruct((n, dout), jnp.float32),
    )(x, W)


def _mid_kernel(acc_ref, hp_ref, di_ref, b_ref, w_ref, o_ref):
    d2 = di_ref[...] * di_ref[...]
    z = jnp.maximum(acc_ref[...] + d2 * hp_ref[...] + b_ref[...], 0.0)
    o_ref[...] = jnp.dot(z, w_ref[...], preferred_element_type=jnp.float32)


def _tc_mid(acc, hp, dinv, b, W):
    n, d = acc.shape
    blk = 1280
    return pl.pallas_call(
        _mid_kernel,
        grid=(n // blk,),
        in_specs=[pl.BlockSpec((blk, d), lambda i: (i, 0)),
                  pl.BlockSpec((blk, d), lambda i: (i, 0)),
                  pl.BlockSpec((blk, 1), lambda i: (i, 0)),
                  pl.BlockSpec((1, d), lambda i: (0, 0)),
                  pl.BlockSpec((d, d), lambda i: (0, 0))],
        out_specs=pl.BlockSpec((blk, d), lambda i: (i, 0)),
        out_shape=jax.ShapeDtypeStruct((n, d), jnp.float32),
    )(acc, hp, dinv, b, W)


def _fin_kernel(acc_ref, hp_ref, di_ref, b_ref, a2_ref, w3_ref, b3_ref,
                o_ref, scr_ref):
    i = pl.program_id(0)
    d2 = di_ref[...] * di_ref[...]
    z = jnp.maximum(acc_ref[...] + d2 * hp_ref[...] + b_ref[...], 0.0)
    pv = jnp.sum(a2_ref[...] * z, axis=0, keepdims=True)

    @pl.when(i == 0)
    def _():
        scr_ref[...] = pv

    @pl.when(i > 0)
    def _():
        scr_ref[...] = scr_ref[...] + pv

    @pl.when(i == pl.num_programs(0) - 1)
    def _():
        logits = jnp.dot(scr_ref[...], w3_ref[...],
                         preferred_element_type=jnp.float32) + b3_ref[...]
        mx = jnp.max(logits)
        ls = logits - mx
        o_ref[...] = ls - jnp.log(jnp.sum(jnp.exp(ls)))


def _tc_fin(acc, hp, dinv, b, a2c, W3, b3):
    n, d = acc.shape
    dout = W3.shape[1]
    blk = 1280
    return pl.pallas_call(
        _fin_kernel,
        grid=(n // blk,),
        in_specs=[pl.BlockSpec((blk, d), lambda i: (i, 0)),
                  pl.BlockSpec((blk, d), lambda i: (i, 0)),
                  pl.BlockSpec((blk, 1), lambda i: (i, 0)),
                  pl.BlockSpec((1, d), lambda i: (0, 0)),
                  pl.BlockSpec((blk, 1), lambda i: (i, 0)),
                  pl.BlockSpec((d, dout), lambda i: (0, 0)),
                  pl.BlockSpec((1, dout), lambda i: (0, 0))],
        out_specs=pl.BlockSpec((1, dout), lambda i: (0, 0)),
        out_shape=jax.ShapeDtypeStruct((1, dout), jnp.float32),
        scratch_shapes=[pltpu.VMEM((1, d), jnp.float32)],
    )(acc, hp, dinv, b, a2c, W3, b3)


def kernel(x, edge_index, edge_weight_params, W1, b1, W2, b2, W3, b3):
    i32 = jnp.int32
    f32 = jnp.float32
    npe = EPAD - E
    pk = edge_index[0].astype(i32) * (M14 + 1) + edge_index[1].astype(i32)
    pk = jnp.concatenate([pk, jnp.zeros((npe,), i32)])
    ewp = jnp.concatenate([edge_weight_params.astype(f32),
                           jnp.full((npe,), -1e9, f32)])
    pk3 = pk.reshape(16 * NCH, 128)
    ewp3 = ewp.reshape(16 * NCH, 128)

    dinv, a2, e1p, e1n, cnt, _m1, _m2 = _prep(pk3, ewp3)
    e1p3 = e1p[:EPAD]
    e1n3 = lax.bitcast_convert_type(e1n[:EPAD], f32)

    xpad = jnp.pad(x, ((0, NPAD - N_NODES), (0, 0)))
    h1p = _tc_matmul(xpad, W1)

    acc1 = _edge_layer(h1p, e1p3, e1n3, cnt, dinv, layer2=False)
    dcol = dinv.reshape(NPAD, 1)
    g2 = _tc_mid(acc1, h1p, dcol, b1.reshape(1, -1), W2)

    acc2 = _edge_layer(g2, e1p3, e1n3, cnt, dinv, layer2=True)
    a2p = a2.at[0].add(dinv[0] * dinv[0])
    logp = _tc_fin(acc2, g2, dcol, b2.reshape(1, -1), a2p.reshape(NPAD, 1),
                   W3, b3.reshape(1, -1))
    return logp[0]
